# baseline (speedup 1.0000x reference)
import functools

import jax
import jax.numpy as jnp
from jax import lax
from jax.experimental import pallas as pl
from jax.experimental.pallas import tpu as pltpu
from jax.experimental.pallas import tpu_sc as plsc

VOCAB = 100000
EMBED = 16
NGRAM = 20
BATCH = 4096
TOK = BATCH * NGRAM
NC = 2
NS = 16
NW = NC * NS
B_PER_W = TOK // NW

TV = 512
GRID_V = (VOCAB + TV - 1) // TV


ROWS_PER_W = BATCH // NW


def _gather_body(table_hbm, idx_hbm, out_hbm, idx_v, rows_v, sem):
    wid = lax.axis_index("s") * NC + lax.axis_index("c")
    base = wid * B_PER_W
    pltpu.sync_copy(idx_hbm.at[pl.ds(base, B_PER_W)], idx_v)
    pltpu.async_copy(table_hbm.at[idx_v], rows_v, sem).wait()
    def _emit_row(r, _):
        pltpu.sync_copy(
            rows_v.at[pl.ds(r * NGRAM, NGRAM)],
            out_hbm.at[wid * ROWS_PER_W + r],
        )
        return ()

    lax.fori_loop(0, ROWS_PER_W, _emit_row, (), unroll=4)


@functools.cache
def _make_gather():
    return pl.kernel(
        _gather_body,
        mesh=plsc.VectorSubcoreMesh(core_axis_name="c", subcore_axis_name="s"),
        out_type=jax.ShapeDtypeStruct((BATCH, NGRAM, EMBED), jnp.float32),
        scratch_types=[
            pltpu.VMEM((B_PER_W,), jnp.int32),
            pltpu.VMEM((B_PER_W, EMBED), jnp.float32),
            pltpu.SemaphoreType.DMA,
        ],
        compiler_params=pltpu.CompilerParams(use_tc_tiling_on_sc=False),
    )


def _logits_body(wT_ref, e_ref, b_ref, o_ref):
    w = wT_ref[...].astype(jnp.bfloat16)
    acc = lax.dot_general(
        w, e_ref[...],
        dimension_numbers=(((0,), (1,)), ((), ())),
        preferred_element_type=jnp.float32,
    )
    o_ref[...] = acc + b_ref[...]


_logits_t = pl.pallas_call(
    _logits_body,
    grid=(GRID_V,),
    in_specs=[
        pl.BlockSpec((NGRAM * EMBED, TV), lambda v: (0, v)),
        pl.BlockSpec((BATCH, NGRAM * EMBED), lambda v: (0, 0)),
        pl.BlockSpec((TV, 1), lambda v: (v, 0)),
    ],
    out_specs=pl.BlockSpec((TV, BATCH), lambda v: (v, 0)),
    out_shape=jax.ShapeDtypeStruct((VOCAB, BATCH), jnp.float32),
    compiler_params=pltpu.CompilerParams(
        dimension_semantics=("arbitrary",),
    ),
)


def kernel(x, embed, fc_w, fc_b):
    x_flat = x.reshape(TOK).astype(jnp.int32)
    emb = _make_gather()(embed, x_flat)
    emb = emb.reshape(BATCH, NGRAM * EMBED).astype(jnp.bfloat16)
    logits_t = _logits_t(fc_w.T, emb, fc_b.reshape(VOCAB, 1))
    return logits_t.T


# speedup vs baseline: 1.0672x; 1.0672x over previous
import functools

import jax
import jax.numpy as jnp
from jax import lax
from jax.experimental import pallas as pl
from jax.experimental.pallas import tpu as pltpu
from jax.experimental.pallas import tpu_sc as plsc

VOCAB = 100000
EMBED = 16
NGRAM = 20
BATCH = 4096
TOK = BATCH * NGRAM
NC = 2
NS = 16
NW = NC * NS
B_PER_W = TOK // NW

TV = 512
GRID_V = (VOCAB + TV - 1) // TV


ROWS_PER_W = BATCH // NW


def _gather_body(table_hbm, idx_hbm, out_hbm, idx_v, rows_v, sem):
    wid = lax.axis_index("s") * NC + lax.axis_index("c")
    base = wid * B_PER_W
    pltpu.sync_copy(idx_hbm.at[pl.ds(base, B_PER_W)], idx_v)
    pltpu.async_copy(table_hbm.at[idx_v], rows_v, sem).wait()
    pltpu.sync_copy(rows_v, out_hbm.at[pl.ds(base, B_PER_W)])


@functools.cache
def _make_gather():
    return pl.kernel(
        _gather_body,
        mesh=plsc.VectorSubcoreMesh(core_axis_name="c", subcore_axis_name="s"),
        out_type=jax.ShapeDtypeStruct((TOK, EMBED), jnp.float32),
        scratch_types=[
            pltpu.VMEM((B_PER_W,), jnp.int32),
            pltpu.VMEM((B_PER_W, EMBED), jnp.float32),
            pltpu.SemaphoreType.DMA,
        ],
        compiler_params=pltpu.CompilerParams(use_tc_tiling_on_sc=False),
    )


def _logits_body(wT_ref, e_ref, b_ref, o_ref, e16_ref):
    @pl.when(pl.program_id(0) == 0)
    def _():
        e16_ref[...] = e_ref[...].astype(jnp.bfloat16)

    w = wT_ref[...].astype(jnp.bfloat16)
    acc = lax.dot_general(
        w, e16_ref[...],
        dimension_numbers=(((0,), (1,)), ((), ())),
        preferred_element_type=jnp.float32,
    )
    o_ref[...] = acc + b_ref[...]


_logits_t = pl.pallas_call(
    _logits_body,
    grid=(GRID_V,),
    in_specs=[
        pl.BlockSpec((NGRAM * EMBED, TV), lambda v: (0, v)),
        pl.BlockSpec((BATCH, NGRAM * EMBED), lambda v: (0, 0)),
        pl.BlockSpec((TV, 1), lambda v: (v, 0)),
    ],
    out_specs=pl.BlockSpec((TV, BATCH), lambda v: (v, 0)),
    out_shape=jax.ShapeDtypeStruct((VOCAB, BATCH), jnp.float32),
    scratch_shapes=[pltpu.VMEM((BATCH, NGRAM * EMBED), jnp.bfloat16)],
    compiler_params=pltpu.CompilerParams(
        dimension_semantics=("arbitrary",),
    ),
)


def kernel(x, embed, fc_w, fc_b):
    x_flat = x.reshape(TOK).astype(jnp.int32)
    emb = _make_gather()(embed, x_flat)
    emb = emb.reshape(BATCH, NGRAM * EMBED)
    logits_t = _logits_t(fc_w.T, emb, fc_b.reshape(VOCAB, 1))
    return logits_t.T


# speedup vs baseline: 1.0706x; 1.0031x over previous
import functools

import jax
import jax.numpy as jnp
from jax import lax
from jax.experimental import pallas as pl
from jax.experimental.pallas import tpu as pltpu
from jax.experimental.pallas import tpu_sc as plsc

VOCAB = 100000
EMBED = 16
NGRAM = 20
BATCH = 4096
TOK = BATCH * NGRAM
NC = 2
NS = 16
NW = NC * NS
B_PER_W = TOK // NW

TV = 512
GRID_V = (VOCAB + TV - 1) // TV


_SEG_TABLE = tuple(
    tuple(
        (p // BATCH, p % BATCH, p - g * B_PER_W,
         min(BATCH - p % BATCH, (g + 1) * B_PER_W - p))
        for p in [g * B_PER_W] + [
            q * BATCH for q in range(20)
            if g * B_PER_W < q * BATCH < (g + 1) * B_PER_W
        ]
    )
    for g in range(8)
)


def _gather_body(table_hbm, xT_hbm, out_hbm, idx_v, rows_v, sem):
    wid = lax.axis_index("s") * NC + lax.axis_index("c")
    g = wid % 8
    jbase = (wid // 8) * 5
    for gg, segs in enumerate(_SEG_TABLE):
        @pl.when(g == gg)
        def _(segs=segs):
            for (jr, b0, off, n) in segs:
                pltpu.sync_copy(
                    xT_hbm.at[jbase + jr, pl.ds(b0, n)],
                    idx_v.at[pl.ds(off, n)],
                )
    pltpu.async_copy(table_hbm.at[idx_v], rows_v, sem).wait()
    for gg, segs in enumerate(_SEG_TABLE):
        @pl.when(g == gg)
        def _(segs=segs):
            for (jr, b0, off, n) in segs:
                pltpu.sync_copy(
                    rows_v.at[pl.ds(off, n)],
                    out_hbm.at[pl.ds(b0, n),
                               pl.ds((jbase + jr) * EMBED, EMBED)],
                )


@functools.cache
def _make_gather():
    return pl.kernel(
        _gather_body,
        mesh=plsc.VectorSubcoreMesh(core_axis_name="c", subcore_axis_name="s"),
        out_type=jax.ShapeDtypeStruct((BATCH, NGRAM * EMBED), jnp.float32),
        scratch_types=[
            pltpu.VMEM((B_PER_W,), jnp.int32),
            pltpu.VMEM((B_PER_W, EMBED), jnp.float32),
            pltpu.SemaphoreType.DMA,
        ],
        compiler_params=pltpu.CompilerParams(use_tc_tiling_on_sc=False),
    )


def _logits_body(wT_ref, e_ref, b_ref, o_ref, e16_ref):
    @pl.when(pl.program_id(0) == 0)
    def _():
        e16_ref[...] = e_ref[...].astype(jnp.bfloat16)

    w = wT_ref[...].astype(jnp.bfloat16)
    acc = lax.dot_general(
        w, e16_ref[...],
        dimension_numbers=(((0,), (1,)), ((), ())),
        preferred_element_type=jnp.float32,
    )
    o_ref[...] = acc + b_ref[...]


_logits_t = pl.pallas_call(
    _logits_body,
    grid=(GRID_V,),
    in_specs=[
        pl.BlockSpec((NGRAM * EMBED, TV), lambda v: (0, v)),
        pl.BlockSpec((BATCH, NGRAM * EMBED), lambda v: (0, 0)),
        pl.BlockSpec((TV, 1), lambda v: (v, 0)),
    ],
    out_specs=pl.BlockSpec((TV, BATCH), lambda v: (v, 0)),
    out_shape=jax.ShapeDtypeStruct((VOCAB, BATCH), jnp.float32),
    scratch_shapes=[pltpu.VMEM((BATCH, NGRAM * EMBED), jnp.bfloat16)],
    compiler_params=pltpu.CompilerParams(
        dimension_semantics=("arbitrary",),
    ),
)


def kernel(x, embed, fc_w, fc_b):
    emb = _make_gather()(embed, x.T.astype(jnp.int32))
    logits_t = _logits_t(fc_w.T, emb, fc_b.reshape(VOCAB, 1))
    return logits_t.T


# speedup vs baseline: 1.1347x; 1.0599x over previous
import functools

import jax
import jax.numpy as jnp
from jax import lax
from jax.experimental import pallas as pl
from jax.experimental.pallas import tpu as pltpu
from jax.experimental.pallas import tpu_sc as plsc

VOCAB = 100000
EMBED = 16
NGRAM = 20
BATCH = 4096
TOK = BATCH * NGRAM
NC = 2
NS = 16
NW = NC * NS
B_PER_W = TOK // NW

TV = 1024
GRID_V = (VOCAB + TV - 1) // TV


_SEG_TABLE = tuple(
    tuple(
        (p // BATCH, p % BATCH, p - g * B_PER_W,
         min(BATCH - p % BATCH, (g + 1) * B_PER_W - p))
        for p in [g * B_PER_W] + [
            q * BATCH for q in range(20)
            if g * B_PER_W < q * BATCH < (g + 1) * B_PER_W
        ]
    )
    for g in range(8)
)


def _gather_body(table_hbm, xT_hbm, out_hbm, idx_v, rows_v, sem):
    wid = lax.axis_index("s") * NC + lax.axis_index("c")
    g = wid % 8
    jbase = (wid // 8) * 5
    for gg, segs in enumerate(_SEG_TABLE):
        @pl.when(g == gg)
        def _(segs=segs):
            for (jr, b0, off, n) in segs:
                pltpu.sync_copy(
                    xT_hbm.at[jbase + jr, pl.ds(b0, n)],
                    idx_v.at[pl.ds(off, n)],
                )
    pltpu.async_copy(table_hbm.at[idx_v], rows_v, sem).wait()
    for gg, segs in enumerate(_SEG_TABLE):
        @pl.when(g == gg)
        def _(segs=segs):
            for (jr, b0, off, n) in segs:
                pltpu.sync_copy(
                    rows_v.at[pl.ds(off, n)],
                    out_hbm.at[pl.ds(b0, n),
                               pl.ds((jbase + jr) * EMBED, EMBED)],
                )


@functools.cache
def _make_gather():
    return pl.kernel(
        _gather_body,
        mesh=plsc.VectorSubcoreMesh(core_axis_name="c", subcore_axis_name="s"),
        out_type=jax.ShapeDtypeStruct((BATCH, NGRAM * EMBED), jnp.float32),
        scratch_types=[
            pltpu.VMEM((B_PER_W,), jnp.int32),
            pltpu.VMEM((B_PER_W, EMBED), jnp.float32),
            pltpu.SemaphoreType.DMA,
        ],
        compiler_params=pltpu.CompilerParams(use_tc_tiling_on_sc=False),
    )


def _logits_body(wT_ref, e_ref, b_ref, o_ref, e16_ref):
    @pl.when(pl.program_id(0) == 0)
    def _():
        e16_ref[...] = e_ref[...].astype(jnp.bfloat16)

    w = wT_ref[...].astype(jnp.bfloat16)
    acc = lax.dot_general(
        w, e16_ref[...],
        dimension_numbers=(((0,), (1,)), ((), ())),
        preferred_element_type=jnp.float32,
    )
    o_ref[...] = acc + b_ref[...]


_logits_t = pl.pallas_call(
    _logits_body,
    grid=(GRID_V,),
    in_specs=[
        pl.BlockSpec((NGRAM * EMBED, TV), lambda v: (0, v)),
        pl.BlockSpec((BATCH, NGRAM * EMBED), lambda v: (0, 0)),
        pl.BlockSpec((TV, 1), lambda v: (v, 0)),
    ],
    out_specs=pl.BlockSpec((TV, BATCH), lambda v: (v, 0)),
    out_shape=jax.ShapeDtypeStruct((VOCAB, BATCH), jnp.float32),
    scratch_shapes=[pltpu.VMEM((BATCH, NGRAM * EMBED), jnp.bfloat16)],
    compiler_params=pltpu.CompilerParams(
        dimension_semantics=("arbitrary",),
    ),
)


def kernel(x, embed, fc_w, fc_b):
    emb = _make_gather()(embed, x.T.astype(jnp.int32))
    logits_t = _logits_t(fc_w.T, emb, fc_b.reshape(VOCAB, 1))
    return logits_t.T


# speedup vs baseline: 1.1458x; 1.0098x over previous
import functools

import jax
import jax.numpy as jnp
from jax import lax
from jax.experimental import pallas as pl
from jax.experimental.pallas import tpu as pltpu
from jax.experimental.pallas import tpu_sc as plsc

VOCAB = 100000
EMBED = 16
NGRAM = 20
BATCH = 4096
TOK = BATCH * NGRAM
NC = 2
NS = 16
NW = NC * NS
B_PER_W = TOK // NW

TV = 1280
GRID_V = (VOCAB + TV - 1) // TV


_SEG_TABLE = tuple(
    tuple(
        (p // BATCH, p % BATCH, p - g * B_PER_W,
         min(BATCH - p % BATCH, (g + 1) * B_PER_W - p))
        for p in [g * B_PER_W] + [
            q * BATCH for q in range(20)
            if g * B_PER_W < q * BATCH < (g + 1) * B_PER_W
        ]
    )
    for g in range(8)
)


def _gather_body(table_hbm, xT_hbm, out_hbm, idx_v, rows_v, sem):
    wid = lax.axis_index("s") * NC + lax.axis_index("c")
    g = wid % 8
    jbase = (wid // 8) * 5
    for gg, segs in enumerate(_SEG_TABLE):
        @pl.when(g == gg)
        def _(segs=segs):
            for (jr, b0, off, n) in segs:
                pltpu.sync_copy(
                    xT_hbm.at[jbase + jr, pl.ds(b0, n)],
                    idx_v.at[pl.ds(off, n)],
                )
    pltpu.async_copy(table_hbm.at[idx_v], rows_v, sem).wait()
    for gg, segs in enumerate(_SEG_TABLE):
        @pl.when(g == gg)
        def _(segs=segs):
            for (jr, b0, off, n) in segs:
                pltpu.sync_copy(
                    rows_v.at[pl.ds(off, n)],
                    out_hbm.at[pl.ds(b0, n),
                               pl.ds((jbase + jr) * EMBED, EMBED)],
                )


@functools.cache
def _make_gather():
    return pl.kernel(
        _gather_body,
        mesh=plsc.VectorSubcoreMesh(core_axis_name="c", subcore_axis_name="s"),
        out_type=jax.ShapeDtypeStruct((BATCH, NGRAM * EMBED), jnp.float32),
        scratch_types=[
            pltpu.VMEM((B_PER_W,), jnp.int32),
            pltpu.VMEM((B_PER_W, EMBED), jnp.float32),
            pltpu.SemaphoreType.DMA,
        ],
        compiler_params=pltpu.CompilerParams(use_tc_tiling_on_sc=False),
    )


def _logits_body(wT_ref, e_ref, b_ref, o_ref, e16_ref):
    @pl.when(pl.program_id(0) == 0)
    def _():
        e16_ref[...] = e_ref[...].astype(jnp.bfloat16)

    w = wT_ref[...].astype(jnp.bfloat16)
    acc = lax.dot_general(
        w, e16_ref[...],
        dimension_numbers=(((0,), (1,)), ((), ())),
        preferred_element_type=jnp.float32,
    )
    o_ref[...] = acc + b_ref[...]


_logits_t = pl.pallas_call(
    _logits_body,
    grid=(GRID_V,),
    in_specs=[
        pl.BlockSpec((NGRAM * EMBED, TV), lambda v: (0, v)),
        pl.BlockSpec((BATCH, NGRAM * EMBED), lambda v: (0, 0)),
        pl.BlockSpec((TV, 1), lambda v: (v, 0)),
    ],
    out_specs=pl.BlockSpec((TV, BATCH), lambda v: (v, 0)),
    out_shape=jax.ShapeDtypeStruct((VOCAB, BATCH), jnp.float32),
    scratch_shapes=[pltpu.VMEM((BATCH, NGRAM * EMBED), jnp.bfloat16)],
    compiler_params=pltpu.CompilerParams(
        dimension_semantics=("arbitrary",),
    ),
)


def kernel(x, embed, fc_w, fc_b):
    emb = _make_gather()(embed, x.T.astype(jnp.int32))
    logits_t = _logits_t(fc_w.T, emb, fc_b.reshape(VOCAB, 1))
    return logits_t.T
